# baseline (speedup 1.0000x reference)
import functools

import jax
import jax.numpy as jnp
from jax import lax
from jax.experimental import pallas as pl
from jax.experimental.pallas import tpu as pltpu
from jax.experimental.pallas import tpu_sc as plsc

N = 10000
E = 320000
F_IN = 128
H = 32
C = 47

NC = 2
NS = 16
NW = NC * NS
CH = 80
K = E // (NW * CH)
GK = 5
DGK = 25
NG = K // GK
N_PAD = 10240
RP = N_PAD // NS

_mesh = plsc.VectorSubcoreMesh(
    core_axis_name="c", subcore_axis_name="s", num_cores=NC, num_subcores=NS)


@functools.partial(
    pl.kernel,
    out_type=jax.ShapeDtypeStruct((NC * N_PAD,), jnp.float32),
    mesh=_mesh,
    compiler_params=pltpu.CompilerParams(use_tc_tiling_on_sc=False),
    scratch_types=[
        pltpu.VMEM((K, CH), jnp.int32),
        pltpu.VMEM((CH,), jnp.float32),
        pltpu.VMEM((RP,), jnp.float32),
        pltpu.VMEM_SHARED((N_PAD,), jnp.float32),
        pltpu.SemaphoreType.DMA,
        pltpu.SemaphoreType.DMA,
    ],
)
def _deg_kernel(dst_hbm, out_hbm, idx_v, ones_v, stage_v, acc_sh, sem, sem_s):
    c = lax.axis_index("c")
    s = lax.axis_index("s")
    wid = s * NC + c
    cp = pltpu.async_copy(dst_hbm.at[pl.ds(wid * K, K)], idx_v, sem)
    for j in range(CH // 16):
        ones_v[pl.ds(j * 16, 16)] = jnp.ones((16,), jnp.float32)
    for j in range(RP // 16):
        stage_v[pl.ds(j * 16, 16)] = jnp.zeros((16,), jnp.float32)
    pltpu.sync_copy(stage_v, acc_sh.at[pl.ds(s * RP, RP)])
    cp.wait()
    plsc.subcore_barrier()

    def body(t, carry):
        cps = [pltpu.async_copy(ones_v, acc_sh.at[idx_v.at[t * DGK + b]],
                                sem_s, add=True) for b in range(DGK)]
        for cp_ in cps:
            cp_.wait()
        return carry

    lax.fori_loop(0, K // DGK, body, 0)
    plsc.subcore_barrier()
    pltpu.sync_copy(acc_sh.at[pl.ds(s * RP, RP)], stage_v)
    pltpu.sync_copy(stage_v, out_hbm.at[pl.ds(c * N_PAD + s * RP, RP)])


@functools.partial(
    pl.kernel,
    out_type=jax.ShapeDtypeStruct((NC * N_PAD, H), jnp.float32),
    mesh=_mesh,
    compiler_params=pltpu.CompilerParams(use_tc_tiling_on_sc=False),
    scratch_types=[
        pltpu.VMEM((K, CH), jnp.int32),
        pltpu.VMEM((K, CH), jnp.int32),
        pltpu.VMEM((2 * GK, CH, H), jnp.float32),
        pltpu.VMEM((RP, H), jnp.float32),
        pltpu.VMEM_SHARED((N_PAD, H), jnp.float32),
        pltpu.VMEM_SHARED((N_PAD, H), jnp.float32),
        pltpu.SemaphoreType.DMA,
        pltpu.SemaphoreType.DMA,
        pltpu.SemaphoreType.DMA,
    ],
)
def _agg_kernel(g_hbm, src_hbm, dst_hbm, out_hbm,
                src_v, dst_v, rows_v, stage_v, acc_sh, tab_sh,
                sem_g, sem_s, sem_i):
    c = lax.axis_index("c")
    s = lax.axis_index("s")
    wid = s * NC + c
    cp0 = pltpu.async_copy(src_hbm.at[pl.ds(wid * K, K)], src_v, sem_i)
    cp1 = pltpu.async_copy(dst_hbm.at[pl.ds(wid * K, K)], dst_v, sem_i)
    pltpu.sync_copy(g_hbm.at[pl.ds(s * RP, RP)], stage_v)
    pltpu.sync_copy(stage_v, tab_sh.at[pl.ds(s * RP, RP)])
    for r in range(16):
        for h2 in range(H // 16):
            stage_v[r, pl.ds(h2 * 16, 16)] = jnp.zeros((16,), jnp.float32)
    zcps = [pltpu.async_copy(stage_v.at[pl.ds(0, 16)],
                             acc_sh.at[pl.ds(s * RP + t * 16, 16)], sem_s)
            for t in range(RP // 16)]
    for cp_ in zcps:
        cp_.wait()
    cp0.wait()
    cp1.wait()
    plsc.subcore_barrier()

    def gathers(g, half):
        for b in range(GK):
            pltpu.async_copy(tab_sh.at[src_v.at[g * GK + b]],
                             rows_v.at[half * GK + b], sem_g)

    gathers(0, 0)
    gathers(1, 1)

    def body(g, carry):
        half = lax.rem(g, 2)
        for b in range(GK):
            pltpu.make_async_copy(tab_sh.at[src_v.at[g * GK + b]],
                                  rows_v.at[half * GK + b], sem_g).wait()
        cps = [pltpu.async_copy(rows_v.at[half * GK + b],
                                acc_sh.at[dst_v.at[g * GK + b]], sem_s, add=True)
               for b in range(GK)]
        for cp_ in cps:
            cp_.wait()

        @pl.when(g + 2 < NG)
        def _():
            gathers(g + 2, half)

        return carry

    lax.fori_loop(0, NG, body, 0)
    plsc.subcore_barrier()
    pltpu.sync_copy(acc_sh.at[pl.ds(s * RP, RP)], stage_v)
    pltpu.sync_copy(stage_v, out_hbm.at[pl.ds(c * N_PAD + s * RP, RP)])


def _tc0_body(x_ref, w1_ref, h1_ref):
    h1_ref[...] = jnp.dot(x_ref[...], w1_ref[...],
                          preferred_element_type=jnp.float32)


_tc0 = pl.pallas_call(
    _tc0_body,
    out_shape=jax.ShapeDtypeStruct((N, H), jnp.float32),
)


def _tc1_body(h1_ref, degp_ref, g1_ref, dinv_ref):
    deg = degp_ref[:N_PAD] + degp_ref[N_PAD:] + 1.0
    dinv = lax.rsqrt(deg)[:, None]
    g1_ref[:N] = h1_ref[...] * dinv[:N]
    g1_ref[N:] = jnp.zeros((N_PAD - N, H), jnp.float32)
    dinv_ref[...] = dinv


_tc1 = pl.pallas_call(
    _tc1_body,
    out_shape=[jax.ShapeDtypeStruct((N_PAD, H), jnp.float32),
               jax.ShapeDtypeStruct((N_PAD, 1), jnp.float32)],
)


def _tc2_body(part_ref, g1_ref, dinv_ref, w2_ref, b1_ref, x1_ref, g2_ref):
    ssum = part_ref[:N_PAD] + part_ref[N_PAD:] + g1_ref[...]
    x1 = jnp.maximum(ssum * dinv_ref[...] + b1_ref[...], 0.0)
    x1_ref[...] = x1
    h2 = jnp.dot(x1, w2_ref[...], preferred_element_type=jnp.float32)
    g2_ref[...] = h2 * dinv_ref[...]


_tc2 = pl.pallas_call(
    _tc2_body,
    out_shape=[jax.ShapeDtypeStruct((N_PAD, H), jnp.float32),
               jax.ShapeDtypeStruct((N_PAD, H), jnp.float32)],
)


def _tc3_body(part_ref, g2_ref, dinv_ref, b2_ref, x1_ref, wl_ref, bl_ref, out_ref):
    ssum = part_ref[:N_PAD] + part_ref[N_PAD:] + g2_ref[...]
    x2 = jnp.maximum(ssum * dinv_ref[...] + b2_ref[...], 0.0)
    logits = (jnp.dot(x1_ref[...], wl_ref[:H], preferred_element_type=jnp.float32)
              + jnp.dot(x2, wl_ref[H:], preferred_element_type=jnp.float32)
              + bl_ref[...])
    z = logits[:N]
    m = jnp.max(z, axis=-1, keepdims=True)
    lse = m + jnp.log(jnp.sum(jnp.exp(z - m), axis=-1, keepdims=True))
    out_ref[...] = z - lse


_tc3 = pl.pallas_call(
    _tc3_body,
    out_shape=jax.ShapeDtypeStruct((N, C), jnp.float32),
)


def kernel(x, edge_index, W1, b1, W2, b2, Wl, bl):
    e2 = edge_index.reshape(2, NW * K, CH)
    srcp = e2[0]
    dstp = e2[1]

    degp = _deg_kernel(dstp)
    h1 = _tc0(x, W1)
    g1, dinv = _tc1(h1, degp)
    part1 = _agg_kernel(g1, srcp, dstp)
    x1, g2 = _tc2(part1, g1, dinv, W2, b1.reshape(1, H))
    part2 = _agg_kernel(g2, srcp, dstp)
    return _tc3(part2, g2, dinv, b2.reshape(1, H), x1, Wl, bl.reshape(1, C))


# speedup vs baseline: 1.0302x; 1.0302x over previous
import functools

import jax
import jax.numpy as jnp
from jax import lax
from jax.experimental import pallas as pl
from jax.experimental.pallas import tpu as pltpu
from jax.experimental.pallas import tpu_sc as plsc

N = 10000
E = 320000
F_IN = 128
H = 32
C = 47

NC = 2
NS = 16
NW = NC * NS
CH = 80
K = E // (NW * CH)
GK = 5
DGK = 25
NG = K // GK
N_PAD = 10240
RP = N_PAD // NS

_mesh = plsc.VectorSubcoreMesh(
    core_axis_name="c", subcore_axis_name="s", num_cores=NC, num_subcores=NS)


@functools.partial(
    pl.kernel,
    out_type=jax.ShapeDtypeStruct((NC * N_PAD,), jnp.float32),
    mesh=_mesh,
    compiler_params=pltpu.CompilerParams(use_tc_tiling_on_sc=False),
    scratch_types=[
        pltpu.VMEM((K, CH), jnp.int32),
        pltpu.VMEM((CH,), jnp.float32),
        pltpu.VMEM((RP,), jnp.float32),
        pltpu.VMEM_SHARED((N_PAD,), jnp.float32),
        pltpu.SemaphoreType.DMA,
        pltpu.SemaphoreType.DMA,
    ],
)
def _deg_kernel(dst_hbm, out_hbm, idx_v, ones_v, stage_v, acc_sh, sem, sem_s):
    c = lax.axis_index("c")
    s = lax.axis_index("s")
    wid = s * NC + c
    cp = pltpu.async_copy(dst_hbm.at[pl.ds(wid * K, K)], idx_v, sem)
    for j in range(CH // 16):
        ones_v[pl.ds(j * 16, 16)] = jnp.ones((16,), jnp.float32)
    for j in range(RP // 16):
        stage_v[pl.ds(j * 16, 16)] = jnp.zeros((16,), jnp.float32)
    pltpu.sync_copy(stage_v, acc_sh.at[pl.ds(s * RP, RP)])
    cp.wait()
    plsc.subcore_barrier()

    def body(t, carry):
        cps = [pltpu.async_copy(ones_v, acc_sh.at[idx_v.at[t * DGK + b]],
                                sem_s, add=True) for b in range(DGK)]
        for cp_ in cps:
            cp_.wait()
        return carry

    lax.fori_loop(0, K // DGK, body, 0)
    plsc.subcore_barrier()
    pltpu.sync_copy(acc_sh.at[pl.ds(s * RP, RP)], stage_v)
    pltpu.sync_copy(stage_v, out_hbm.at[pl.ds(c * N_PAD + s * RP, RP)])


@functools.partial(
    pl.kernel,
    out_type=jax.ShapeDtypeStruct((NC * N_PAD, H), jnp.float32),
    mesh=_mesh,
    compiler_params=pltpu.CompilerParams(use_tc_tiling_on_sc=False),
    scratch_types=[
        pltpu.VMEM((K, CH), jnp.int32),
        pltpu.VMEM((K, CH), jnp.int32),
        pltpu.VMEM((2 * GK, CH, H), jnp.float32),
        pltpu.VMEM((RP, H), jnp.float32),
        pltpu.VMEM_SHARED((N_PAD, H), jnp.float32),
        pltpu.VMEM_SHARED((N_PAD, H), jnp.float32),
        pltpu.SemaphoreType.DMA,
        pltpu.SemaphoreType.DMA,
        pltpu.SemaphoreType.DMA,
    ],
)
def _agg_kernel(g_hbm, src_hbm, dst_hbm, out_hbm,
                src_v, dst_v, rows_v, stage_v, acc_sh, tab_sh,
                sem_g, sem_s, sem_i):
    c = lax.axis_index("c")
    s = lax.axis_index("s")
    wid = s * NC + c
    cp0 = pltpu.async_copy(src_hbm.at[pl.ds(wid * K, K)], src_v, sem_i)
    cp1 = pltpu.async_copy(dst_hbm.at[pl.ds(wid * K, K)], dst_v, sem_i)
    pltpu.sync_copy(g_hbm.at[pl.ds(s * RP, RP)], stage_v)
    pltpu.sync_copy(stage_v, tab_sh.at[pl.ds(s * RP, RP)])
    for r in range(16):
        for h2 in range(H // 16):
            stage_v[r, pl.ds(h2 * 16, 16)] = jnp.zeros((16,), jnp.float32)
    zcps = [pltpu.async_copy(stage_v.at[pl.ds(0, 16)],
                             acc_sh.at[pl.ds(s * RP + t * 16, 16)], sem_s)
            for t in range(RP // 16)]
    for cp_ in zcps:
        cp_.wait()
    cp0.wait()
    cp1.wait()
    plsc.subcore_barrier()

    def gathers(g, half):
        for b in range(GK):
            pltpu.async_copy(tab_sh.at[src_v.at[g * GK + b]],
                             rows_v.at[half * GK + b], sem_g)

    gathers(0, 0)
    gathers(1, 1)

    def body(g, carry):
        half = lax.rem(g, 2)
        for b in range(GK):
            pltpu.make_async_copy(tab_sh.at[src_v.at[g * GK + b]],
                                  rows_v.at[half * GK + b], sem_g).wait()
        cps = [pltpu.async_copy(rows_v.at[half * GK + b],
                                acc_sh.at[dst_v.at[g * GK + b]], sem_s, add=True)
               for b in range(GK)]
        for cp_ in cps:
            cp_.wait()

        @pl.when(g + 2 < NG)
        def _():
            gathers(g + 2, half)

        return carry

    lax.fori_loop(0, NG, body, 0)
    plsc.subcore_barrier()
    pltpu.sync_copy(acc_sh.at[pl.ds(s * RP, RP)], stage_v)
    pltpu.sync_copy(stage_v, out_hbm.at[pl.ds(c * N_PAD + s * RP, RP)])


def _tc1_body(x_ref, w1_ref, degp_ref, g1_ref, dinv_ref):
    deg = degp_ref[:N_PAD] + degp_ref[N_PAD:] + 1.0
    dinv = lax.rsqrt(deg)[:, None]
    h1 = jnp.dot(x_ref[...], w1_ref[...], preferred_element_type=jnp.float32)
    g1_ref[:N] = h1 * dinv[:N]
    g1_ref[N:] = jnp.zeros((N_PAD - N, H), jnp.float32)
    dinv_ref[...] = dinv


_tc1 = pl.pallas_call(
    _tc1_body,
    out_shape=[jax.ShapeDtypeStruct((N_PAD, H), jnp.float32),
               jax.ShapeDtypeStruct((N_PAD, 1), jnp.float32)],
)


def _tc2_body(part_ref, g1_ref, dinv_ref, w2_ref, b1_ref, x1_ref, g2_ref):
    ssum = part_ref[:N_PAD] + part_ref[N_PAD:] + g1_ref[...]
    x1 = jnp.maximum(ssum * dinv_ref[...] + b1_ref[...], 0.0)
    x1_ref[...] = x1
    h2 = jnp.dot(x1, w2_ref[...], preferred_element_type=jnp.float32)
    g2_ref[...] = h2 * dinv_ref[...]


_tc2 = pl.pallas_call(
    _tc2_body,
    out_shape=[jax.ShapeDtypeStruct((N_PAD, H), jnp.float32),
               jax.ShapeDtypeStruct((N_PAD, H), jnp.float32)],
)


def _tc3_body(part_ref, g2_ref, dinv_ref, b2_ref, x1_ref, wl_ref, bl_ref, out_ref):
    ssum = part_ref[:N_PAD] + part_ref[N_PAD:] + g2_ref[...]
    x2 = jnp.maximum(ssum * dinv_ref[...] + b2_ref[...], 0.0)
    logits = (jnp.dot(x1_ref[...], wl_ref[:H], preferred_element_type=jnp.float32)
              + jnp.dot(x2, wl_ref[H:], preferred_element_type=jnp.float32)
              + bl_ref[...])
    z = logits[:N]
    m = jnp.max(z, axis=-1, keepdims=True)
    lse = m + jnp.log(jnp.sum(jnp.exp(z - m), axis=-1, keepdims=True))
    out_ref[...] = z - lse


_tc3 = pl.pallas_call(
    _tc3_body,
    out_shape=jax.ShapeDtypeStruct((N, C), jnp.float32),
)


def kernel(x, edge_index, W1, b1, W2, b2, Wl, bl):
    e2 = edge_index.reshape(2, NW * K, CH)
    srcp = e2[0]
    dstp = e2[1]

    degp = _deg_kernel(dstp)
    g1, dinv = _tc1(x, W1, degp)
    part1 = _agg_kernel(g1, srcp, dstp)
    x1, g2 = _tc2(part1, g1, dinv, W2, b1.reshape(1, H))
    part2 = _agg_kernel(g2, srcp, dstp)
    return _tc3(part2, g2, dinv, b2.reshape(1, H), x1, Wl, bl.reshape(1, C))
